# baseline (speedup 1.0000x reference)
import functools

import jax
import jax.numpy as jnp
from jax import lax
from jax.experimental import pallas as pl
from jax.experimental.pallas import tpu as pltpu
from jax.experimental.pallas import tpu_sc as plsc

D_MODEL = 4096
N_EXP = 64
K = 8
T_BLK = 1024

NC, NS, L = 2, 16, 16
NW = NC * NS


def _matmul_body(x_ref, wt_ref, logits_ref, logits_t_ref):
    logits = jnp.dot(x_ref[...], wt_ref[...], preferred_element_type=jnp.float32)
    logits_ref[...] = logits
    logits_t_ref[...] = logits.T


def _gating_body(lt_hbm, idx_t_hbm, gate_t_hbm, lv, idx_v, gate_v, sem):
    n_tok = lt_hbm.shape[1]
    t_per_w = n_tok // NW
    wid = lax.axis_index("s") * NC + lax.axis_index("c")
    base = wid * t_per_w

    pltpu.async_copy(lt_hbm.at[:, pl.ds(base, t_per_w)], lv, sem).wait()

    neg_inf = jnp.full((L,), -jnp.inf, dtype=jnp.float32)

    def group(g, _):
        col = g * L

        vals = [neg_inf] * K
        idxs = [jnp.zeros((L,), jnp.int32)] * K
        for e in range(N_EXP):
            t = lv[e, pl.ds(col, L)]
            ti = jnp.full((L,), e, jnp.int32)
            for j in range(K):
                c = (t > vals[j]) | ((t == vals[j]) & (ti < idxs[j]))
                nv = jnp.where(c, t, vals[j])
                nt = jnp.where(c, vals[j], t)
                ni = jnp.where(c, ti, idxs[j])
                nti = jnp.where(c, idxs[j], ti)
                vals[j], t = nv, nt
                idxs[j], ti = ni, nti

        ev = [jnp.exp(v - vals[0]) for v in vals]
        s = ev[0]
        for j in range(1, K):
            s = s + ev[j]
        inv = 1.0 / s

        for j in range(K):
            idx_v[j, pl.ds(col, L)] = idxs[j]
            gate_v[j, pl.ds(col, L)] = ev[j] * inv
        return ()

    lax.fori_loop(0, t_per_w // L, group, (), unroll=False)

    pltpu.sync_copy(idx_v, idx_t_hbm.at[:, pl.ds(base, t_per_w)])
    pltpu.sync_copy(gate_v, gate_t_hbm.at[:, pl.ds(base, t_per_w)])


@jax.jit
def kernel(x, router_weights):
    b, s, d = x.shape
    n_tok = b * s
    x2 = x.reshape(n_tok, d)
    wt = router_weights.T

    logits, logits_t = pl.pallas_call(
        _matmul_body,
        grid=(n_tok // T_BLK,),
        in_specs=[
            pl.BlockSpec((T_BLK, d), lambda i: (i, 0)),
            pl.BlockSpec((d, N_EXP), lambda i: (0, 0)),
        ],
        out_specs=[
            pl.BlockSpec((T_BLK, N_EXP), lambda i: (i, 0)),
            pl.BlockSpec((N_EXP, T_BLK), lambda i: (0, i)),
        ],
        out_shape=[
            jax.ShapeDtypeStruct((n_tok, N_EXP), jnp.float32),
            jax.ShapeDtypeStruct((N_EXP, n_tok), jnp.float32),
        ],
    )(x2, wt)

    t_per_w = n_tok // NW
    gating = pl.kernel(
        _gating_body,
        out_type=[
            jax.ShapeDtypeStruct((K, n_tok), jnp.int32),
            jax.ShapeDtypeStruct((K, n_tok), jnp.float32),
        ],
        mesh=plsc.VectorSubcoreMesh(
            core_axis_name="c", subcore_axis_name="s", num_cores=NC, num_subcores=NS
        ),
        scratch_types=[
            pltpu.VMEM((N_EXP, t_per_w), jnp.float32),
            pltpu.VMEM((K, t_per_w), jnp.int32),
            pltpu.VMEM((K, t_per_w), jnp.float32),
            pltpu.SemaphoreType.DMA,
        ],
    )
    idx_t, gate_t = gating(logits_t)

    return (
        idx_t.T.reshape(b, s, K),
        gate_t.T.reshape(b, s, K),
        logits.reshape(b, s, N_EXP),
    )


# speedup vs baseline: 1.1490x; 1.1490x over previous
import functools

import jax
import jax.numpy as jnp
from jax.experimental import pallas as pl

D_MODEL = 4096
N_EXP = 64
K = 8
T_BLK = 256


def _router_body(x_ref, wt_ref, idx_ref, gate_ref, logits_ref):
    logits = jnp.dot(x_ref[...], wt_ref[...], preferred_element_type=jnp.float32)
    logits_ref[...] = logits

    lt = logits.T
    iota = jax.lax.broadcasted_iota(jnp.int32, lt.shape, 0).astype(jnp.float32)
    cur = lt
    vals = []
    idxs = []
    for _ in range(K):
        m = jnp.max(cur, axis=0, keepdims=True)
        amax = jnp.min(
            jnp.where(cur == m, iota, jnp.float32(N_EXP)), axis=0, keepdims=True
        )
        vals.append(m)
        idxs.append(amax)
        cur = jnp.where(iota == amax, -jnp.inf, cur)

    tv = jnp.concatenate(vals, axis=0)
    ti = jnp.concatenate(idxs, axis=0)
    ev = jnp.exp(tv - tv[0:1, :])
    g = ev / jnp.sum(ev, axis=0, keepdims=True)
    gate_ref[...] = g.T
    idx_ref[...] = ti.T.astype(jnp.int32)


@jax.jit
def kernel(x, router_weights):
    b, s, d = x.shape
    n_tok = b * s
    x2 = x.reshape(n_tok, d)
    wt = router_weights.T

    grid = (n_tok // T_BLK,)
    idx_out, gates, logits = pl.pallas_call(
        _router_body,
        grid=grid,
        in_specs=[
            pl.BlockSpec((T_BLK, d), lambda i: (i, 0)),
            pl.BlockSpec((d, N_EXP), lambda i: (0, 0)),
        ],
        out_specs=[
            pl.BlockSpec((T_BLK, K), lambda i: (i, 0)),
            pl.BlockSpec((T_BLK, K), lambda i: (i, 0)),
            pl.BlockSpec((T_BLK, N_EXP), lambda i: (i, 0)),
        ],
        out_shape=[
            jax.ShapeDtypeStruct((n_tok, K), jnp.int32),
            jax.ShapeDtypeStruct((n_tok, K), jnp.float32),
            jax.ShapeDtypeStruct((n_tok, N_EXP), jnp.float32),
        ],
    )(x2, wt)

    return (
        idx_out.reshape(b, s, K),
        gates.reshape(b, s, K),
        logits.reshape(b, s, N_EXP),
    )
